# baseline (speedup 1.0000x reference)
import functools
import jax
import jax.numpy as jnp
from jax import lax
from jax.experimental import pallas as pl
from jax.experimental.pallas import tpu as pltpu
from jax.experimental.pallas import tpu_sc as plsc

_B, _N = 16, 8192
_IN, _H, _EMB, _HA, _OUT = 64, 128, 64, 128, 64
_NUM_AISLES = 32
_G = 2
_ROWS = _B * _N
_NSEG = _B * _NUM_AISLES
_NC, _NS = 2, 16
_HSEG = _NSEG // _NC
_RPW = _ROWS // _NS
_CHUNK = 512
_NCHUNK = _RPW // _CHUNK


def _lrelu(v):
    return jnp.maximum(v, v * jnp.asarray(0.01, v.dtype))


def _mlp1_kernel(x_ref, w1_ref, b1_ref, w2_ref, b2_ref, w3_ref, b3_ref, z_ref):
    f32, bf = jnp.float32, jnp.bfloat16
    w1 = w1_ref[...].astype(bf)
    w2 = w2_ref[...].astype(bf)
    w3 = w3_ref[...].astype(bf)
    xb = x_ref[...].astype(bf)
    h = _lrelu(jnp.dot(xb, w1, preferred_element_type=f32).astype(bf)
               + b1_ref[...].astype(bf)[None, :])
    h = _lrelu(jnp.dot(h, w2, preferred_element_type=f32).astype(bf)
               + b2_ref[...].astype(bf)[None, :])
    z_ref[...] = (jnp.dot(h, w3, preferred_element_type=f32)
                  + b3_ref[...][None, :])


def _sc_segsum(z_hbm, ids2_hbm, zsums_hbm, sums_hbm,
               idx_v, z_v, sums_sp):
    c = lax.axis_index("c")
    s = lax.axis_index("s")

    @pl.when(s == 0)
    def _zero():
        pltpu.sync_copy(zsums_hbm, sums_sp)

    plsc.subcore_barrier()
    for k in range(_NCHUNK):
        base = s * _RPW + k * _CHUNK
        pltpu.sync_copy(ids2_hbm.at[c].at[pl.ds(base, _CHUNK)], idx_v)
        pltpu.sync_copy(z_hbm.at[pl.ds(base, _CHUNK), :], z_v)
        pltpu.sync_copy(z_v, sums_sp.at[idx_v], add=True)
    plsc.subcore_barrier()
    seg = _HSEG // _NS
    pltpu.sync_copy(sums_sp.at[pl.ds(s * seg, seg), :],
                    sums_hbm.at[c].at[pl.ds(s * seg, seg), :])


def _mlp2_kernel(z_ref, ids_ref, mask_ref, sums_ref,
                 w4_ref, b4_ref, w5_ref, b5_ref, w6_ref, b6_ref, out_ref):
    f32, bf = jnp.float32, jnp.bfloat16
    b = pl.program_id(0)
    nrow = _G * _N
    ncls = _G * _NUM_AISLES
    w5 = w5_ref[...].astype(bf)
    w6 = w6_ref[...].astype(bf)
    zb = z_ref[...].astype(bf)

    ids = ids_ref[...][None, :]
    ids = ids + _NUM_AISLES * (
        jax.lax.broadcasted_iota(jnp.int32, (1, nrow), 1) // _N)
    oh = (jnp.broadcast_to(ids, (ncls, nrow)) ==
          jax.lax.broadcasted_iota(jnp.int32, (ncls, nrow), 0)).astype(bf)

    sums = sums_ref[...]
    counts = jnp.sum(oh.astype(f32), axis=1, keepdims=True)
    means = (sums / jnp.maximum(counts, 1.0)).astype(bf)
    g = jax.lax.dot_general(oh, means, (((0,), (0,)), ((), ())),
                            preferred_element_type=f32)

    cat = jnp.concatenate([zb, g.astype(bf)], axis=1)
    h2 = _lrelu(jnp.dot(cat, w4_ref[...].astype(bf), preferred_element_type=f32).astype(bf)
                + b4_ref[...].astype(bf)[None, :])
    h2 = _lrelu(jnp.dot(h2, w5, preferred_element_type=f32).astype(bf)
                + b5_ref[...].astype(bf)[None, :])
    scores = jax.lax.dot_general(w6, h2, (((0,), (1,)), ((), ())),
                                 preferred_element_type=f32) + b6_ref[0]

    for gi in range(_G):
        row = _G * b + gi
        sl = scores[:, gi * _N:(gi + 1) * _N]
        mk = mask_ref[pl.ds(row, 1), :]
        logits = jnp.where(mk != 0, sl, -jnp.inf)
        mx = jnp.max(logits, axis=1, keepdims=True)
        e = jnp.exp(logits - mx)
        out_ref[pl.ds(row, 1), :] = e / jnp.sum(e, axis=1, keepdims=True)


def kernel(x, aisle_nrs, mask, W1, b1, W2, b2, W3, b3, W4, b4, W5, b5, W6, b6):
    f32 = jnp.float32
    ids = aisle_nrs.astype(jnp.int32)
    ids_global = ids + _NUM_AISLES * (
        jnp.arange(_ROWS, dtype=jnp.int32) // _N)
    ids2 = jnp.stack([
        jnp.where((ids_global >= c * _HSEG) & (ids_global < (c + 1) * _HSEG),
                  ids_global - c * _HSEG, _HSEG)
        for c in range(_NC)])

    full = lambda arr: pl.BlockSpec(arr.shape, lambda b: (0,) * arr.ndim)

    z = pl.pallas_call(
        _mlp1_kernel,
        grid=(_B // _G,),
        in_specs=[pl.BlockSpec((_G * _N, _IN), lambda b: (b, 0))]
                 + [full(w) for w in (W1, b1, W2, b2, W3, b3)],
        out_specs=pl.BlockSpec((_G * _N, _EMB), lambda b: (b, 0)),
        out_shape=jax.ShapeDtypeStruct((_ROWS, _EMB), f32),
        compiler_params=pltpu.CompilerParams(
            dimension_semantics=("arbitrary",)),
    )(x, W1, b1, W2, b2, W3, b3)

    mesh = plsc.VectorSubcoreMesh(core_axis_name="c", subcore_axis_name="s")
    sc = functools.partial(
        pl.kernel, mesh=mesh,
        out_type=jax.ShapeDtypeStruct((_NC, _HSEG, _EMB), f32),
        scratch_types=[
            pltpu.VMEM((_CHUNK,), jnp.int32),
            pltpu.VMEM((_CHUNK, _EMB), f32),
            pltpu.VMEM_SHARED((_HSEG + 1, _EMB), f32),
        ],
    )(_sc_segsum)
    zsums = jnp.zeros((_HSEG + 1, _EMB), f32)
    sums = sc(z, ids2, zsums).reshape(_NSEG, _EMB)

    probs = pl.pallas_call(
        _mlp2_kernel,
        grid=(_B // _G,),
        in_specs=[pl.BlockSpec((_G * _N, _EMB), lambda b: (b, 0)),
                  pl.BlockSpec((_G * _N,), lambda b: (b,)),
                  full(mask),
                  pl.BlockSpec((_G * _NUM_AISLES, _EMB), lambda b: (b, 0))]
                 + [full(w) for w in (W4, b4, W5, b5, W6, b6)],
        out_specs=pl.BlockSpec((_B, _N), lambda b: (0, 0)),
        out_shape=jax.ShapeDtypeStruct((_B, _N), f32),
        compiler_params=pltpu.CompilerParams(
            dimension_semantics=("arbitrary",)),
    )(z, ids, mask, sums, W4, b4, W5, b5, W6, b6)

    return probs


# speedup vs baseline: 2.1317x; 2.1317x over previous
import jax
import jax.numpy as jnp
from jax.experimental import pallas as pl
from jax.experimental.pallas import tpu as pltpu

_B, _N = 16, 8192
_IN, _H, _EMB, _HA, _OUT = 64, 128, 64, 128, 64
_NUM_AISLES = 32
_G = 4


def _lrelu(v):
    return jnp.maximum(v, v * jnp.asarray(0.01, v.dtype))


def _fused_kernel(x_ref, ids_ref, mask_ref,
                  w1_ref, b1_ref, w2_ref, b2_ref, w3_ref, b3_ref,
                  w4_ref, b4_ref, w5_ref, b5_ref, w6_ref, b6_ref,
                  out_ref):
    f32, bf = jnp.float32, jnp.bfloat16
    b = pl.program_id(0)
    w1 = w1_ref[...].astype(bf)
    w2 = w2_ref[...].astype(bf)
    w3 = w3_ref[...].astype(bf)
    w4 = w4_ref[...].astype(bf)
    w5 = w5_ref[...].astype(bf)
    w6 = w6_ref[...].astype(bf)
    xb = x_ref[...].astype(bf)
    h = _lrelu(jnp.dot(xb, w1, preferred_element_type=f32).astype(bf)
               + b1_ref[...].astype(bf)[None, :])
    h = _lrelu(jnp.dot(h, w2, preferred_element_type=f32).astype(bf)
               + b2_ref[...].astype(bf)[None, :])
    zb = (jnp.dot(h, w3, preferred_element_type=f32).astype(bf)
          + b3_ref[...].astype(bf)[None, :])

    nrow = _G * _N
    ncls = _G * _NUM_AISLES
    ids = ids_ref[...][None, :]
    ids = ids + _NUM_AISLES * (
        jax.lax.broadcasted_iota(jnp.int32, (1, nrow), 1) // _N)
    oh = (jnp.broadcast_to(ids, (ncls, nrow)) ==
          jax.lax.broadcasted_iota(jnp.int32, (ncls, nrow), 0)).astype(bf)
    z1 = jnp.concatenate([zb, jnp.ones((nrow, 1), bf)], axis=1)
    sc = jax.lax.dot_general(oh, z1, (((1,), (0,)), ((), ())),
                             preferred_element_type=f32)
    sums, counts = sc[:, :_EMB], sc[:, _EMB:]
    means = (sums / jnp.maximum(counts, 1.0)).astype(bf)
    g = jax.lax.dot_general(oh, means, (((0,), (0,)), ((), ())),
                            preferred_element_type=f32)

    cat = jnp.concatenate([zb, g.astype(bf)], axis=1)
    h2 = _lrelu(jnp.dot(cat, w4, preferred_element_type=f32).astype(bf)
                + b4_ref[...].astype(bf)[None, :])
    h2 = _lrelu(jnp.dot(h2, w5, preferred_element_type=f32).astype(bf)
                + b5_ref[...].astype(bf)[None, :])
    scores = jax.lax.dot_general(w6, h2, (((0,), (1,)), ((), ())),
                                 preferred_element_type=f32) + b6_ref[0]

    for gi in range(_G):
        row = _G * b + gi
        s = scores[:, gi * _N:(gi + 1) * _N]
        mk = mask_ref[pl.ds(row, 1), :]
        logits = jnp.where(mk != 0, s, -jnp.inf)
        mx = jnp.max(logits, axis=1, keepdims=True)
        e = jnp.exp(logits - mx)
        out_ref[pl.ds(row, 1), :] = e / jnp.sum(e, axis=1, keepdims=True)


def kernel(x, aisle_nrs, mask, W1, b1, W2, b2, W3, b3, W4, b4, W5, b5, W6, b6):
    ids = aisle_nrs.astype(jnp.int32)

    full = lambda arr: pl.BlockSpec(arr.shape, lambda b: (0,) * arr.ndim)
    weights = [W1, b1, W2, b2, W3, b3, W4, b4, W5, b5, W6, b6]

    probs = pl.pallas_call(
        _fused_kernel,
        grid=(_B // _G,),
        in_specs=[pl.BlockSpec((_G * _N, _IN), lambda b: (b, 0)),
                  pl.BlockSpec((_G * _N,), lambda b: (b,)),
                  full(mask)] + [full(w) for w in weights],
        out_specs=pl.BlockSpec((_B, _N), lambda b: (0, 0)),
        out_shape=jax.ShapeDtypeStruct((_B, _N), jnp.float32),
        compiler_params=pltpu.CompilerParams(
            dimension_semantics=("arbitrary",)),
    )(x, ids, mask, *weights)

    return probs
